# baseline (speedup 1.0000x reference)
import functools

import jax
import jax.numpy as jnp
from jax import lax
from jax.experimental import pallas as pl
from jax.experimental.pallas import tpu as pltpu
from jax.experimental.pallas import tpu_sc as plsc


def _emb_kernel(n_per_w, ch, num_cores, d):
    steps = n_per_w // ch
    mesh = plsc.VectorSubcoreMesh(core_axis_name="c", subcore_axis_name="s")

    def body(idx_hbm, table_hbm, out_hbm, idx_v, rows_v, sem):
        wid = lax.axis_index("s") * num_cores + lax.axis_index("c")
        base = wid * n_per_w

        def step(i, carry):
            off = base + i * ch
            pltpu.sync_copy(idx_hbm.at[pl.ds(off, ch)], idx_v)
            pltpu.async_copy(table_hbm.at[idx_v], rows_v, sem).wait()
            pltpu.sync_copy(rows_v, out_hbm.at[pl.ds(off, ch)])
            return carry

        lax.fori_loop(0, steps, step, 0)

    return body, mesh


def kernel(tokens, table):
    b, h = tokens.shape
    v, d = table.shape
    n = b * h
    idx = tokens.reshape(n).astype(jnp.int32)

    info = plsc.get_sparse_core_info()
    nw = info.num_cores * info.num_subcores
    n_per_w = n // nw
    ch = 512

    body, mesh = _emb_kernel(n_per_w, ch, info.num_cores, d)
    emb = functools.partial(
        pl.kernel,
        mesh=mesh,
        out_type=jax.ShapeDtypeStruct((n, d), jnp.float32),
        scratch_types=[
            pltpu.VMEM((ch,), jnp.int32),
            pltpu.VMEM((ch, d), jnp.float32),
            pltpu.SemaphoreType.DMA,
        ],
        compiler_params=pltpu.CompilerParams(use_tc_tiling_on_sc=False),
    )(body)

    out = emb(idx, table)
    return out.reshape(b, h, d)


# speedup vs baseline: 1.0674x; 1.0674x over previous
import functools

import jax
import jax.numpy as jnp
from jax import lax
from jax.experimental import pallas as pl
from jax.experimental.pallas import tpu as pltpu
from jax.experimental.pallas import tpu_sc as plsc

_NBUF = 2


def _emb_body(n_per_w, ch, steps, num_cores):
    nb = _NBUF

    def body(idx_hbm, table_hbm, out_hbm, *refs):
        idx_v = refs[0:nb]
        rows_v = refs[nb : 2 * nb]
        sem_i = refs[2 * nb : 3 * nb]
        sem_g = refs[3 * nb : 4 * nb]
        sem_o = refs[4 * nb : 5 * nb]
        wid = lax.axis_index("s") * num_cores + lax.axis_index("c")
        base = wid * n_per_w

        def idx_copy(i, b):
            return pltpu.make_async_copy(
                idx_hbm.at[pl.ds(base + i * ch, ch)], idx_v[b], sem_i[b]
            )

        def gather_copy(b):
            return pltpu.make_async_copy(table_hbm.at[idx_v[b]], rows_v[b], sem_g[b])

        def out_copy(i, b):
            return pltpu.make_async_copy(
                rows_v[b], out_hbm.at[pl.ds(base + i * ch, ch)], sem_o[b]
            )

        for b in range(nb):
            idx_copy(b, b).start()

        def group(g, carry):
            i0 = g * nb
            for b in range(nb):
                i = i0 + b
                idx_copy(i, b).wait()

                @pl.when(i >= nb)
                def _():
                    out_copy(i - nb, b).wait()

                gather_copy(b).start()
            for b in range(nb):
                i = i0 + b
                gather_copy(b).wait()

                @pl.when(i + nb < steps)
                def _():
                    idx_copy(i + nb, b).start()

                out_copy(i, b).start()
            return carry

        lax.fori_loop(0, steps // nb, group, 0)

        for b in range(nb):
            out_copy(steps - nb + b, b).wait()

    return body


def kernel(tokens, table):
    b, h = tokens.shape
    v, d = table.shape
    n = b * h
    idx = tokens.reshape(n).astype(jnp.int32)

    info = plsc.get_sparse_core_info()
    nw = info.num_cores * info.num_subcores
    n_per_w = n // nw
    ch = 512
    steps = n_per_w // ch

    emb = functools.partial(
        pl.kernel,
        mesh=plsc.VectorSubcoreMesh(core_axis_name="c", subcore_axis_name="s"),
        out_type=jax.ShapeDtypeStruct((n, d), jnp.float32),
        scratch_types=(
            [pltpu.VMEM((ch,), jnp.int32) for _ in range(_NBUF)]
            + [pltpu.VMEM((ch, d), jnp.float32) for _ in range(_NBUF)]
            + [pltpu.SemaphoreType.DMA for _ in range(3 * _NBUF)]
        ),
        compiler_params=pltpu.CompilerParams(use_tc_tiling_on_sc=False),
    )(_emb_body(n_per_w, ch, steps, info.num_cores))

    out = emb(idx, table)
    return out.reshape(b, h, d)


# speedup vs baseline: 1.0690x; 1.0015x over previous
import functools

import jax
import jax.numpy as jnp
from jax import lax
from jax.experimental import pallas as pl
from jax.experimental.pallas import tpu as pltpu
from jax.experimental.pallas import tpu_sc as plsc

_NBUF = 2
_RC = 4


def _emb_body(rows_per_w, steps, num_cores):
    nb = _NBUF
    rc = _RC

    def body(tok_hbm, table_hbm, out_hbm, *refs):
        idx_v = refs[0:nb]
        rows_v = refs[nb : 2 * nb]
        sem_i = refs[2 * nb : 3 * nb]
        sem_g = refs[3 * nb : 4 * nb]
        sem_o = refs[4 * nb : 5 * nb]
        wid = lax.axis_index("s") * num_cores + lax.axis_index("c")
        row0 = wid * rows_per_w

        def idx_copy(i, b):
            return pltpu.make_async_copy(
                tok_hbm.at[pl.ds(row0 + i * rc, rc)], idx_v[b], sem_i[b]
            )

        def gather_copy(b, j):
            return pltpu.make_async_copy(
                table_hbm.at[idx_v[b].at[j]], rows_v[b].at[j], sem_g[b]
            )

        def out_copy(i, b):
            return pltpu.make_async_copy(
                rows_v[b], out_hbm.at[pl.ds(row0 + i * rc, rc)], sem_o[b]
            )

        for b in range(nb):
            idx_copy(b, b).start()

        def group(g, carry):
            i0 = g * nb
            for b in range(nb):
                i = i0 + b
                idx_copy(i, b).wait()

                @pl.when(i >= nb)
                def _():
                    out_copy(i - nb, b).wait()

                for j in range(rc):
                    gather_copy(b, j).start()
            for b in range(nb):
                i = i0 + b
                for j in range(rc):
                    gather_copy(b, j).wait()

                @pl.when(i + nb < steps)
                def _():
                    idx_copy(i + nb, b).start()

                out_copy(i, b).start()
            return carry

        lax.fori_loop(0, steps // nb, group, 0)

        for b in range(nb):
            out_copy(steps - nb + b, b).wait()

    return body


def kernel(tokens, table):
    b, h = tokens.shape
    v, d = table.shape
    tok = tokens.astype(jnp.int32)

    info = plsc.get_sparse_core_info()
    nw = info.num_cores * info.num_subcores
    rows_per_w = b // nw
    steps = rows_per_w // _RC

    emb = functools.partial(
        pl.kernel,
        mesh=plsc.VectorSubcoreMesh(core_axis_name="c", subcore_axis_name="s"),
        out_type=jax.ShapeDtypeStruct((b, h, d), jnp.float32),
        scratch_types=(
            [pltpu.VMEM((_RC, h), jnp.int32) for _ in range(_NBUF)]
            + [pltpu.VMEM((_RC, h, d), jnp.float32) for _ in range(_NBUF)]
            + [pltpu.SemaphoreType.DMA for _ in range(3 * _NBUF)]
        ),
        compiler_params=pltpu.CompilerParams(use_tc_tiling_on_sc=False),
    )(_emb_body(rows_per_w, steps, info.num_cores))

    return emb(tok, table)


# speedup vs baseline: 1.8286x; 1.7106x over previous
import functools

import jax
import jax.numpy as jnp
from jax import lax
from jax.experimental import pallas as pl
from jax.experimental.pallas import tpu as pltpu
from jax.experimental.pallas import tpu_sc as plsc

_NBUF = 4
_CH = 400


def _emb_body(n_per_w, steps, num_cores, d):
    nb = _NBUF
    ch = _CH

    def body(idx_hbm, table_hbm, out_hbm, *refs):
        idx_v = refs[0:nb]
        rows_v = refs[nb : 2 * nb]
        sem_i = refs[2 * nb : 3 * nb]
        sem_g = refs[3 * nb : 4 * nb]
        sem_o = refs[4 * nb : 5 * nb]
        wid = lax.axis_index("s") * num_cores + lax.axis_index("c")
        base = wid * n_per_w

        def idx_copy(i, b):
            return pltpu.make_async_copy(
                idx_hbm.at[pl.ds(base + i * ch, ch)], idx_v[b], sem_i[b]
            )

        def gather_copy(b):
            return pltpu.make_async_copy(table_hbm.at[idx_v[b]], rows_v[b], sem_g[b])

        def out_copy(i, b):
            return pltpu.make_async_copy(
                rows_v[b],
                out_hbm.at[pl.ds(base + i * ch, ch), pl.ds(0, d)],
                sem_o[b],
            )

        for b in range(nb):
            idx_copy(b, b).start()

        def group(g, carry):
            i0 = g * nb
            for b in range(nb):
                i = i0 + b
                idx_copy(i, b).wait()

                @pl.when(i >= nb)
                def _():
                    out_copy(i - nb, b).wait()

                gather_copy(b).start()
            for b in range(nb):
                i = i0 + b
                gather_copy(b).wait()

                @pl.when(i + nb < steps)
                def _():
                    idx_copy(i + nb, b).start()

                out_copy(i, b).start()
            return carry

        lax.fori_loop(0, steps // nb, group, 0)

        for b in range(nb):
            out_copy(steps - nb + b, b).wait()

    return body


def kernel(tokens, table):
    b, h = tokens.shape
    v, d = table.shape
    n = b * h
    idx = tokens.reshape(n).astype(jnp.int32) * 2
    tab2 = jnp.pad(table, ((0, 0), (0, 128 - d))).reshape(2 * v, d)

    info = plsc.get_sparse_core_info()
    nw = info.num_cores * info.num_subcores
    n_per_w = n // nw
    steps = n_per_w // _CH

    emb = functools.partial(
        pl.kernel,
        mesh=plsc.VectorSubcoreMesh(core_axis_name="c", subcore_axis_name="s"),
        out_type=jax.ShapeDtypeStruct((n, 128), jnp.float32),
        scratch_types=(
            [pltpu.VMEM((_CH,), jnp.int32) for _ in range(_NBUF)]
            + [pltpu.VMEM((_CH, d), jnp.float32) for _ in range(_NBUF)]
            + [pltpu.SemaphoreType.DMA for _ in range(3 * _NBUF)]
        ),
        compiler_params=pltpu.CompilerParams(use_tc_tiling_on_sc=False),
    )(_emb_body(n_per_w, steps, info.num_cores, d))

    return emb(idx, tab2)[:, :d].reshape(b, h, d)
